# baseline (speedup 1.0000x reference)
import functools

import jax
import jax.numpy as jnp
from jax import lax
from jax.experimental import pallas as pl
from jax.experimental.pallas import tpu as pltpu
from jax.experimental.pallas import tpu_sc as plsc

_NC = 2
_NS = 16
_NW = _NC * _NS
_B = 80

_mesh = plsc.VectorSubcoreMesh(
    core_axis_name="c", subcore_axis_name="s", num_cores=_NC, num_subcores=_NS
)


def _split_rows(n):
    a = ((n // _NS) // 8) * 8
    return a, n - (_NS - 1) * a


def _part_copy(src, dst, s, n, dst_off=0):
    a, tail = _split_rows(n)

    @pl.when(s < _NS - 1)
    def _():
        pltpu.sync_copy(src.at[pl.ds(s * a, a)],
                        dst.at[pl.ds(dst_off + s * a, a)])

    @pl.when(s == _NS - 1)
    def _():
        pltpu.sync_copy(src.at[pl.ds((_NS - 1) * a, tail)],
                        dst.at[pl.ds(dst_off + (_NS - 1) * a, tail)])


def _sc_gather(h, idx):
    tot = idx.shape[0]
    n, d = h.shape
    per_w = tot // _NW
    steps = per_w // _B
    pairs = steps // 2

    @functools.partial(
        pl.kernel,
        out_type=jax.ShapeDtypeStruct((tot, d), jnp.float32),
        mesh=_mesh,
        scratch_types=[
            pltpu.VMEM((per_w,), jnp.int32),
            pltpu.VMEM((_B, d), jnp.float32),
            pltpu.VMEM((_B, d), jnp.float32),
            pltpu.VMEM_SHARED((n, d), jnp.float32),
            pltpu.SemaphoreType.DMA,
            pltpu.SemaphoreType.DMA,
            pltpu.SemaphoreType.DMA,
            pltpu.SemaphoreType.DMA,
        ],
    )
    def k(h_hbm, idx_hbm, out_hbm, idx_all, r0, r1, h_sp, sg0, sg1, so0, so1):
        c = lax.axis_index("c")
        s = lax.axis_index("s")
        wid = s * _NC + c
        _part_copy(h_hbm, h_sp, s, n)
        base0 = wid * per_w
        pltpu.sync_copy(idx_hbm.at[pl.ds(base0, per_w)], idx_all)
        plsc.subcore_barrier()

        def gath(step, buf, sem):
            pltpu.async_copy(
                h_sp.at[idx_all.at[pl.ds(step * _B, _B)]], buf, sem)

        def gath_wait(buf, sem):
            pltpu.make_async_copy(
                h_sp.at[idx_all.at[pl.ds(0, _B)]], buf, sem).wait()

        def wb(step, buf, sem):
            pltpu.async_copy(buf, out_hbm.at[pl.ds(base0 + step * _B, _B)],
                             sem)

        def wb_wait(buf, sem):
            pltpu.make_async_copy(
                buf, out_hbm.at[pl.ds(base0, _B)], sem).wait()

        gath(0, r0, sg0)

        @pl.loop(0, pairs)
        def _(i):
            @pl.when(i > 0)
            def _():
                wb_wait(r1, so1)

            gath(2 * i + 1, r1, sg1)
            gath_wait(r0, sg0)
            wb(2 * i, r0, so0)
            gath_wait(r1, sg1)
            wb(2 * i + 1, r1, so1)

            @pl.when(i < pairs - 1)
            def _():
                wb_wait(r0, so0)
                gath(2 * i + 2, r0, sg0)

        wb_wait(r0, so0)
        wb_wait(r1, so1)

    return k(h, idx)


def _sc_scatter_m(m, dst2d, zn, chunk_row0):
    e = m.shape[0]
    n = zn.shape[0]
    per_c = e // _NC
    per_w = per_c // _NS
    steps = per_w // _B
    pairs = steps // 2

    idx_rows = ((steps + 7) // 8) * 8 + 8

    @functools.partial(
        pl.kernel,
        out_type=jax.ShapeDtypeStruct((_NC * n, 128), jnp.float32),
        mesh=_mesh,
        scratch_types=[
            pltpu.VMEM((idx_rows, _B), jnp.int32),
            pltpu.VMEM((_B, 128), jnp.float32),
            pltpu.VMEM((_B, 128), jnp.float32),
            pltpu.VMEM_SHARED((n, 128), jnp.float32),
            pltpu.SemaphoreType.DMA,
            pltpu.SemaphoreType.DMA,
        ],
    )
    def k(m_hbm, dst2_hbm, zn_hbm, out_hbm, idx2, r0, r1, acc, sl0, sl1):
        c = lax.axis_index("c")
        s = lax.axis_index("s")
        _part_copy(zn_hbm, acc, s, n)
        row0 = chunk_row0 + c * (per_c // _B) + s * steps
        aligned0 = pl.multiple_of((row0 // 8) * 8, 8)
        delta = row0 - aligned0
        pltpu.sync_copy(dst2_hbm.at[pl.ds(aligned0, idx_rows)], idx2)
        plsc.subcore_barrier()
        base0 = c * per_c + s * per_w

        def ld(j, buf, sem):
            pltpu.async_copy(m_hbm.at[pl.ds(base0 + j * _B, _B)], buf, sem)

        def ld_wait(buf, sem):
            pltpu.make_async_copy(
                m_hbm.at[pl.ds(base0, _B)], buf, sem).wait()

        def sc(j, buf):
            pltpu.sync_copy(buf, acc.at[idx2.at[delta + j]], add=True)

        ld(0, r0, sl0)

        @pl.loop(0, pairs)
        def _(i):
            ld(2 * i + 1, r1, sl1)
            ld_wait(r0, sl0)
            sc(2 * i, r0)

            @pl.when(2 * i + 2 < steps)
            def _():
                ld(2 * i + 2, r0, sl0)

            ld_wait(r1, sl1)
            sc(2 * i + 1, r1)

        if steps % 2:
            ld_wait(r0, sl0)
            sc(steps - 1, r0)
        plsc.subcore_barrier()
        _part_copy(acc, out_hbm, s, n, dst_off=c * n)

    return k(m, dst2d, zn)


def _sc_cnt(dst2d, ones_rows, zn, rows_real):
    n = zn.shape[0]
    per_c_rows = rows_real // _NC
    steps = per_c_rows // _NS
    idx_rows = ((steps + 7) // 8) * 8 + 8

    @functools.partial(
        pl.kernel,
        out_type=jax.ShapeDtypeStruct((_NC * n, 128), jnp.float32),
        mesh=_mesh,
        scratch_types=[
            pltpu.VMEM((idx_rows, _B), jnp.int32),
            pltpu.VMEM((_B, 128), jnp.float32),
            pltpu.VMEM_SHARED((n, 128), jnp.float32),
        ],
    )
    def k(dst2_hbm, ones_hbm, zn_hbm, out_hbm, idx2, ones_v, acc):
        c = lax.axis_index("c")
        s = lax.axis_index("s")
        _part_copy(zn_hbm, acc, s, n)
        pltpu.sync_copy(ones_hbm, ones_v)
        row0 = c * per_c_rows + s * steps
        aligned0 = pl.multiple_of((row0 // 8) * 8, 8)
        delta = row0 - aligned0
        pltpu.sync_copy(dst2_hbm.at[pl.ds(aligned0, idx_rows)], idx2)
        plsc.subcore_barrier()

        @pl.loop(0, steps)
        def _(j):
            pltpu.sync_copy(ones_v, acc.at[idx2.at[delta + j]], add=True)

        plsc.subcore_barrier()
        _part_copy(acc, out_hbm, s, n, dst_off=c * n)

    return k(dst2d, ones_rows, zn)


def _gelu(x):
    return 0.5 * x * (1.0 + lax.erf(x * 0.7071067811865476))


def _edge_body(hs_ref, hd_ref, ef_ref, w1a_ref, w1b_ref, w1c_ref, b1_ref,
               w2_ref, m_ref):
    pre = (
        jnp.dot(hs_ref[...], w1a_ref[...], preferred_element_type=jnp.float32)
        + jnp.dot(hd_ref[...], w1b_ref[...], preferred_element_type=jnp.float32)
        + jnp.dot(ef_ref[...], w1c_ref[...], preferred_element_type=jnp.float32)
        + b1_ref[...]
    )
    m_ref[...] = jnp.dot(_gelu(pre), w2_ref[...],
                         preferred_element_type=jnp.float32)


def _tc_edge(hg, ef, w1a, w1b, w1c, b1, w2):
    e = ef.shape[0]
    be = 512
    nb = e // be

    return pl.pallas_call(
        _edge_body,
        grid=(nb,),
        in_specs=[
            pl.BlockSpec((be, 128), lambda i: (i, 0)),
            pl.BlockSpec((be, 128), lambda i, _nb=nb: (i + _nb, 0)),
            pl.BlockSpec((be, 16), lambda i: (i, 0)),
            pl.BlockSpec((128, 256), lambda i: (0, 0)),
            pl.BlockSpec((128, 256), lambda i: (0, 0)),
            pl.BlockSpec((16, 256), lambda i: (0, 0)),
            pl.BlockSpec((1, 256), lambda i: (0, 0)),
            pl.BlockSpec((256, 128), lambda i: (0, 0)),
        ],
        out_specs=pl.BlockSpec((be, 128), lambda i: (i, 0)),
        out_shape=jax.ShapeDtypeStruct((e, 128), jnp.float32),
        compiler_params=pltpu.CompilerParams(
            dimension_semantics=("parallel",),
        ),
    )(hg, hg, ef, w1a, w1b, w1c, b1, w2)


def _node_body(*refs, nparts):
    h_ref = refs[0]
    ps = refs[1:1 + 2 * nparts]
    cs = refs[1 + 2 * nparts:3 + 2 * nparts]
    u1a_ref, u1b_ref, c1_ref, u2_ref, c2_ref, b2_ref, gamma_ref, beta_ref = \
        refs[3 + 2 * nparts:11 + 2 * nparts]
    out_ref = refs[11 + 2 * nparts]

    h = h_ref[...]
    sm = ps[0][...]
    for p in ps[1:]:
        sm = sm + p[...]
    cnt = cs[0][...][:, 0:1] + cs[1][...][:, 0:1]
    denom = jnp.maximum(cnt, 1.0)
    gate = (cnt > 0).astype(jnp.float32)
    agg = sm / denom + b2_ref[...] * gate
    x2 = (
        jnp.dot(h, u1a_ref[...], preferred_element_type=jnp.float32)
        + jnp.dot(agg, u1b_ref[...], preferred_element_type=jnp.float32)
        + c1_ref[...]
    )
    u = jnp.dot(_gelu(x2), u2_ref[...], preferred_element_type=jnp.float32)
    x = u + c2_ref[...] + h
    mu = jnp.mean(x, axis=1, keepdims=True)
    var = jnp.mean((x - mu) ** 2, axis=1, keepdims=True)
    out_ref[...] = (x - mu) / jnp.sqrt(var + 1e-5) * gamma_ref[...] + beta_ref[...]


def _tc_node(h, parts, cnt2, u1a, u1b, c1, u2, c2, b2, gamma, beta):
    n = h.shape[0]
    bn = 400
    nb = n // bn

    def blk(i):
        return (i, 0)

    def blk_hi(i, _nb=nb):
        return (i + _nb, 0)

    def full(i):
        return (0, 0)

    part_specs = []
    part_args = []
    for p in list(parts) + [cnt2]:
        part_specs.append(pl.BlockSpec((bn, 128), blk))
        part_specs.append(pl.BlockSpec((bn, 128), blk_hi))
        part_args.append(p)
        part_args.append(p)

    return pl.pallas_call(
        functools.partial(_node_body, nparts=len(parts)),
        grid=(nb,),
        in_specs=[pl.BlockSpec((bn, 128), blk)] + part_specs + [
            pl.BlockSpec((128, 256), full),
            pl.BlockSpec((128, 256), full),
            pl.BlockSpec((1, 256), full),
            pl.BlockSpec((256, 128), full),
            pl.BlockSpec((1, 128), full),
            pl.BlockSpec((1, 128), full),
            pl.BlockSpec((1, 128), full),
            pl.BlockSpec((1, 128), full),
        ],
        out_specs=pl.BlockSpec((bn, 128), blk),
        out_shape=jax.ShapeDtypeStruct((n, 128), jnp.float32),
        compiler_params=pltpu.CompilerParams(
            dimension_semantics=("parallel",),
        ),
    )(h, *part_args, u1a, u1b, c1, u2, c2, b2, gamma, beta)


def kernel(node_features, edge_index, edge_features,
           W1, b1, W2, b2, U1, c1, U2, c2, gamma, beta):
    n, d = node_features.shape
    src = edge_index[0]
    dst = edge_index[1]
    e = src.shape[0]

    w1a = W1[:d]
    w1b = W1[d:2 * d]
    w1c = W1[2 * d:]
    u1a = U1[:d]
    u1b = U1[d:]
    b1r = b1.reshape(1, -1)

    unit = _NC * _NS * _B
    chunk_units = [16, 16, 16, 16, 16, 16, 16, 13]
    assert sum(chunk_units) * unit == e

    idx_chunks = []
    off = 0
    offs = []
    for u in chunk_units:
        ec_k = u * unit
        offs.append(off)
        idx_chunks.append(src[off:off + ec_k])
        idx_chunks.append(dst[off:off + ec_k])
        off += ec_k
    idx = jnp.concatenate(idx_chunks)

    dst2d = jnp.concatenate(
        [dst.reshape(-1, _B), jnp.zeros((8, _B), jnp.int32)], axis=0)

    ones_rows = jnp.ones((_B, 128), jnp.float32)
    zn = jnp.zeros((n, 128), jnp.float32)

    cnt2 = _sc_cnt(dst2d, ones_rows, zn, e // _B)

    parts = []
    for k, u in enumerate(chunk_units):
        ec_k = u * unit
        off = offs[k]
        idx_k = idx[2 * off:2 * off + 2 * ec_k]
        hg_k = _sc_gather(node_features, idx_k)
        m_k = _tc_edge(hg_k, edge_features[off:off + ec_k],
                       w1a, w1b, w1c, b1r, W2)
        parts.append(_sc_scatter_m(m_k, dst2d, zn,
                                   chunk_row0=off // _B))

    return _tc_node(node_features, parts, cnt2, u1a, u1b, c1.reshape(1, -1),
                    U2, c2.reshape(1, -1), b2.reshape(1, -1),
                    gamma.reshape(1, -1), beta.reshape(1, -1))


# speedup vs baseline: 1.0169x; 1.0169x over previous
import functools

import jax
import jax.numpy as jnp
from jax import lax
from jax.experimental import pallas as pl
from jax.experimental.pallas import tpu as pltpu
from jax.experimental.pallas import tpu_sc as plsc

_NC = 2
_NS = 16
_NW = _NC * _NS
_B = 80

_mesh = plsc.VectorSubcoreMesh(
    core_axis_name="c", subcore_axis_name="s", num_cores=_NC, num_subcores=_NS
)


def _split_rows(n):
    a = ((n // _NS) // 8) * 8
    return a, n - (_NS - 1) * a


def _part_copy(src, dst, s, n, dst_off=0):
    a, tail = _split_rows(n)

    @pl.when(s < _NS - 1)
    def _():
        pltpu.sync_copy(src.at[pl.ds(s * a, a)],
                        dst.at[pl.ds(dst_off + s * a, a)])

    @pl.when(s == _NS - 1)
    def _():
        pltpu.sync_copy(src.at[pl.ds((_NS - 1) * a, tail)],
                        dst.at[pl.ds(dst_off + (_NS - 1) * a, tail)])


def _sc_gather(h, idx):
    tot = idx.shape[0]
    n, d = h.shape
    per_w = tot // _NW
    steps = per_w // _B
    nbuf = 4
    outer = steps // nbuf
    tail = steps - outer * nbuf

    @functools.partial(
        pl.kernel,
        out_type=jax.ShapeDtypeStruct((tot, d), jnp.float32),
        mesh=_mesh,
        scratch_types=[
            pltpu.VMEM((per_w,), jnp.int32),
            [pltpu.VMEM((_B, d), jnp.float32) for _ in range(nbuf)],
            pltpu.VMEM_SHARED((n, d), jnp.float32),
            [pltpu.SemaphoreType.DMA for _ in range(nbuf)],
            [pltpu.SemaphoreType.DMA for _ in range(nbuf)],
        ],
    )
    def k(h_hbm, idx_hbm, out_hbm, idx_all, bufs, h_sp, sgs, sos):
        c = lax.axis_index("c")
        s = lax.axis_index("s")
        wid = s * _NC + c
        _part_copy(h_hbm, h_sp, s, n)
        base0 = wid * per_w
        pltpu.sync_copy(idx_hbm.at[pl.ds(base0, per_w)], idx_all)
        plsc.subcore_barrier()

        def gath(step, j):
            pltpu.async_copy(
                h_sp.at[idx_all.at[pl.ds(step * _B, _B)]], bufs[j], sgs[j])

        def gath_wait(j):
            pltpu.make_async_copy(
                h_sp.at[idx_all.at[pl.ds(0, _B)]], bufs[j], sgs[j]).wait()

        def wb(step, j):
            pltpu.async_copy(
                bufs[j], out_hbm.at[pl.ds(base0 + step * _B, _B)], sos[j])

        def wb_wait(j):
            pltpu.make_async_copy(
                bufs[j], out_hbm.at[pl.ds(base0, _B)], sos[j]).wait()

        for j in range(nbuf - 1):
            gath(j, j)

        @pl.loop(0, outer)
        def _(i):
            for j in range(nbuf):
                t = i * nbuf + j
                gath_wait(j)
                wb(t, j)
                nxt = (j + nbuf - 1) % nbuf

                @pl.when(t + nbuf - 1 < steps)
                def _(t=t, j=j, nxt=nxt):
                    if j == 0:
                        @pl.when(t >= 1)
                        def _():
                            wb_wait(nxt)
                    else:
                        wb_wait(nxt)
                    gath(t + nbuf - 1, nxt)

        for j in range(tail):
            t = outer * nbuf + j
            gath_wait(j)
            wb(t, j)
            nxt = (j + nbuf - 1) % nbuf
            if t + nbuf - 1 < steps:
                if t >= 1:
                    wb_wait(nxt)
                gath(t + nbuf - 1, nxt)

        for j in range(nbuf):
            wb_wait((steps - 1 - j) % nbuf)

    return k(h, idx)


def _sc_scatter_m(m, dst2d, zn, chunk_row0):
    e = m.shape[0]
    n = zn.shape[0]
    per_c = e // _NC
    per_w = per_c // _NS
    steps = per_w // _B
    pairs = steps // 2

    idx_rows = ((steps + 7) // 8) * 8 + 8

    @functools.partial(
        pl.kernel,
        out_type=jax.ShapeDtypeStruct((_NC * n, 128), jnp.float32),
        mesh=_mesh,
        scratch_types=[
            pltpu.VMEM((idx_rows, _B), jnp.int32),
            pltpu.VMEM((_B, 128), jnp.float32),
            pltpu.VMEM((_B, 128), jnp.float32),
            pltpu.VMEM_SHARED((n, 128), jnp.float32),
            pltpu.SemaphoreType.DMA,
            pltpu.SemaphoreType.DMA,
        ],
    )
    def k(m_hbm, dst2_hbm, zn_hbm, out_hbm, idx2, r0, r1, acc, sl0, sl1):
        c = lax.axis_index("c")
        s = lax.axis_index("s")
        _part_copy(zn_hbm, acc, s, n)
        row0 = chunk_row0 + c * (per_c // _B) + s * steps
        aligned0 = pl.multiple_of((row0 // 8) * 8, 8)
        delta = row0 - aligned0
        pltpu.sync_copy(dst2_hbm.at[pl.ds(aligned0, idx_rows)], idx2)
        plsc.subcore_barrier()
        base0 = c * per_c + s * per_w

        def ld(j, buf, sem):
            pltpu.async_copy(m_hbm.at[pl.ds(base0 + j * _B, _B)], buf, sem)

        def ld_wait(buf, sem):
            pltpu.make_async_copy(
                m_hbm.at[pl.ds(base0, _B)], buf, sem).wait()

        def sc(j, buf):
            pltpu.sync_copy(buf, acc.at[idx2.at[delta + j]], add=True)

        ld(0, r0, sl0)

        @pl.loop(0, pairs)
        def _(i):
            ld(2 * i + 1, r1, sl1)
            ld_wait(r0, sl0)
            sc(2 * i, r0)

            @pl.when(2 * i + 2 < steps)
            def _():
                ld(2 * i + 2, r0, sl0)

            ld_wait(r1, sl1)
            sc(2 * i + 1, r1)

        if steps % 2:
            ld_wait(r0, sl0)
            sc(steps - 1, r0)
        plsc.subcore_barrier()
        _part_copy(acc, out_hbm, s, n, dst_off=c * n)

    return k(m, dst2d, zn)


def _sc_cnt(dst2d, ones_rows, zn, rows_real):
    n = zn.shape[0]
    per_c_rows = rows_real // _NC
    steps = per_c_rows // _NS
    idx_rows = ((steps + 7) // 8) * 8 + 8

    @functools.partial(
        pl.kernel,
        out_type=jax.ShapeDtypeStruct((_NC * n, 128), jnp.float32),
        mesh=_mesh,
        scratch_types=[
            pltpu.VMEM((idx_rows, _B), jnp.int32),
            pltpu.VMEM((_B, 128), jnp.float32),
            pltpu.VMEM_SHARED((n, 128), jnp.float32),
        ],
    )
    def k(dst2_hbm, ones_hbm, zn_hbm, out_hbm, idx2, ones_v, acc):
        c = lax.axis_index("c")
        s = lax.axis_index("s")
        _part_copy(zn_hbm, acc, s, n)
        pltpu.sync_copy(ones_hbm, ones_v)
        row0 = c * per_c_rows + s * steps
        aligned0 = pl.multiple_of((row0 // 8) * 8, 8)
        delta = row0 - aligned0
        pltpu.sync_copy(dst2_hbm.at[pl.ds(aligned0, idx_rows)], idx2)
        plsc.subcore_barrier()

        @pl.loop(0, steps)
        def _(j):
            pltpu.sync_copy(ones_v, acc.at[idx2.at[delta + j]], add=True)

        plsc.subcore_barrier()
        _part_copy(acc, out_hbm, s, n, dst_off=c * n)

    return k(dst2d, ones_rows, zn)


def _gelu(x):
    return 0.5 * x * (1.0 + lax.erf(x * 0.7071067811865476))


def _edge_body(hs_ref, hd_ref, ef_ref, w1a_ref, w1b_ref, w1c_ref, b1_ref,
               w2_ref, m_ref):
    pre = (
        jnp.dot(hs_ref[...], w1a_ref[...], preferred_element_type=jnp.float32)
        + jnp.dot(hd_ref[...], w1b_ref[...], preferred_element_type=jnp.float32)
        + jnp.dot(ef_ref[...], w1c_ref[...], preferred_element_type=jnp.float32)
        + b1_ref[...]
    )
    m_ref[...] = jnp.dot(_gelu(pre), w2_ref[...],
                         preferred_element_type=jnp.float32)


def _tc_edge(hg, ef, w1a, w1b, w1c, b1, w2):
    e = ef.shape[0]
    be = 512
    nb = e // be

    return pl.pallas_call(
        _edge_body,
        grid=(nb,),
        in_specs=[
            pl.BlockSpec((be, 128), lambda i: (i, 0)),
            pl.BlockSpec((be, 128), lambda i, _nb=nb: (i + _nb, 0)),
            pl.BlockSpec((be, 16), lambda i: (i, 0)),
            pl.BlockSpec((128, 256), lambda i: (0, 0)),
            pl.BlockSpec((128, 256), lambda i: (0, 0)),
            pl.BlockSpec((16, 256), lambda i: (0, 0)),
            pl.BlockSpec((1, 256), lambda i: (0, 0)),
            pl.BlockSpec((256, 128), lambda i: (0, 0)),
        ],
        out_specs=pl.BlockSpec((be, 128), lambda i: (i, 0)),
        out_shape=jax.ShapeDtypeStruct((e, 128), jnp.float32),
        compiler_params=pltpu.CompilerParams(
            dimension_semantics=("parallel",),
        ),
    )(hg, hg, ef, w1a, w1b, w1c, b1, w2)


def _node_body(*refs, nparts):
    h_ref = refs[0]
    ps = refs[1:1 + 2 * nparts]
    cs = refs[1 + 2 * nparts:3 + 2 * nparts]
    u1a_ref, u1b_ref, c1_ref, u2_ref, c2_ref, b2_ref, gamma_ref, beta_ref = \
        refs[3 + 2 * nparts:11 + 2 * nparts]
    out_ref = refs[11 + 2 * nparts]

    h = h_ref[...]
    sm = ps[0][...]
    for p in ps[1:]:
        sm = sm + p[...]
    cnt = cs[0][...][:, 0:1] + cs[1][...][:, 0:1]
    denom = jnp.maximum(cnt, 1.0)
    gate = (cnt > 0).astype(jnp.float32)
    agg = sm / denom + b2_ref[...] * gate
    x2 = (
        jnp.dot(h, u1a_ref[...], preferred_element_type=jnp.float32)
        + jnp.dot(agg, u1b_ref[...], preferred_element_type=jnp.float32)
        + c1_ref[...]
    )
    u = jnp.dot(_gelu(x2), u2_ref[...], preferred_element_type=jnp.float32)
    x = u + c2_ref[...] + h
    mu = jnp.mean(x, axis=1, keepdims=True)
    var = jnp.mean((x - mu) ** 2, axis=1, keepdims=True)
    out_ref[...] = (x - mu) / jnp.sqrt(var + 1e-5) * gamma_ref[...] + beta_ref[...]


def _tc_node(h, parts, cnt2, u1a, u1b, c1, u2, c2, b2, gamma, beta):
    n = h.shape[0]
    bn = 400
    nb = n // bn

    def blk(i):
        return (i, 0)

    def blk_hi(i, _nb=nb):
        return (i + _nb, 0)

    def full(i):
        return (0, 0)

    part_specs = []
    part_args = []
    for p in list(parts) + [cnt2]:
        part_specs.append(pl.BlockSpec((bn, 128), blk))
        part_specs.append(pl.BlockSpec((bn, 128), blk_hi))
        part_args.append(p)
        part_args.append(p)

    return pl.pallas_call(
        functools.partial(_node_body, nparts=len(parts)),
        grid=(nb,),
        in_specs=[pl.BlockSpec((bn, 128), blk)] + part_specs + [
            pl.BlockSpec((128, 256), full),
            pl.BlockSpec((128, 256), full),
            pl.BlockSpec((1, 256), full),
            pl.BlockSpec((256, 128), full),
            pl.BlockSpec((1, 128), full),
            pl.BlockSpec((1, 128), full),
            pl.BlockSpec((1, 128), full),
            pl.BlockSpec((1, 128), full),
        ],
        out_specs=pl.BlockSpec((bn, 128), blk),
        out_shape=jax.ShapeDtypeStruct((n, 128), jnp.float32),
        compiler_params=pltpu.CompilerParams(
            dimension_semantics=("parallel",),
        ),
    )(h, *part_args, u1a, u1b, c1, u2, c2, b2, gamma, beta)


def kernel(node_features, edge_index, edge_features,
           W1, b1, W2, b2, U1, c1, U2, c2, gamma, beta):
    n, d = node_features.shape
    src = edge_index[0]
    dst = edge_index[1]
    e = src.shape[0]

    w1a = W1[:d]
    w1b = W1[d:2 * d]
    w1c = W1[2 * d:]
    u1a = U1[:d]
    u1b = U1[d:]
    b1r = b1.reshape(1, -1)

    unit = _NC * _NS * _B
    chunk_units = [25, 25, 25, 25, 25]
    assert sum(chunk_units) * unit == e

    idx_chunks = []
    off = 0
    offs = []
    for u in chunk_units:
        ec_k = u * unit
        offs.append(off)
        idx_chunks.append(src[off:off + ec_k])
        idx_chunks.append(dst[off:off + ec_k])
        off += ec_k
    idx = jnp.concatenate(idx_chunks)

    dst2d = jnp.concatenate(
        [dst.reshape(-1, _B), jnp.zeros((8, _B), jnp.int32)], axis=0)

    ones_rows = jnp.ones((_B, 128), jnp.float32)
    zn = jnp.zeros((n, 128), jnp.float32)

    cnt2 = _sc_cnt(dst2d, ones_rows, zn, e // _B)

    parts = []
    for k, u in enumerate(chunk_units):
        ec_k = u * unit
        off = offs[k]
        idx_k = idx[2 * off:2 * off + 2 * ec_k]
        hg_k = _sc_gather(node_features, idx_k)
        m_k = _tc_edge(hg_k, edge_features[off:off + ec_k],
                       w1a, w1b, w1c, b1r, W2)
        parts.append(_sc_scatter_m(m_k, dst2d, zn,
                                   chunk_row0=off // _B))

    return _tc_node(node_features, parts, cnt2, u1a, u1b, c1.reshape(1, -1),
                    U2, c2.reshape(1, -1), b2.reshape(1, -1),
                    gamma.reshape(1, -1), beta.reshape(1, -1))
